# baseline (speedup 1.0000x reference)
import jax
import jax.numpy as jnp
from jax.experimental import pallas as pl
from jax.experimental.pallas import tpu as pltpu

_BB = 256


def _vq_block(x_ref, a_ref, ml_ref, mask_ref, logp_k_ref, a2_ref):
    bb, d = x_ref.shape
    k = a_ref.shape[1]

    @pl.when(pl.program_id(0) == 0)
    def _():
        a0 = a_ref[0]
        a2_ref[...] = jnp.sum(a0 * a0, axis=1)[None, :]

    x = x_ref[...]
    a = a_ref[0]
    xa = jax.lax.dot_general(x * -2.0, a, (((1,), (1,)), ((), ())),
                             preferred_element_type=jnp.float32)
    x2 = jnp.sum(x * x, axis=1, keepdims=True)
    dists = (x2 + a2_ref[...]) + xa
    minv = jnp.min(dists, axis=1, keepdims=True)
    eq = dists == minv
    ml = ml_ref[...]
    shifted = ml - jnp.max(ml)
    lp = shifted - jnp.log(jnp.sum(jnp.exp(shifted)))
    mask_ref[...] = eq.astype(jnp.int8)
    logp_k_ref[...] = jnp.sum(jnp.where(eq, lp, 0.0), axis=1, keepdims=True)
    nsel = jnp.count_nonzero(eq)

    @pl.when(nsel != bb)
    def _fix():
        iota = jax.lax.broadcasted_iota(jnp.int32, (bb, k), 1)
        idx = jnp.min(jnp.where(eq, iota, k), axis=1, keepdims=True)
        m = iota == idx
        mask_ref[...] = m.astype(jnp.int8)
        logp_k_ref[...] = jnp.sum(jnp.where(m, lp, 0.0), axis=1, keepdims=True)


def kernel(x, logp, anchor_pts, mixture_logits):
    b, d = x.shape[0], x.size // x.shape[0]
    xf = x.reshape(b, d)
    k = anchor_pts.shape[1]
    ml2 = mixture_logits.reshape(1, k)
    bb = _BB if b % _BB == 0 else b
    mask, logp_k = pl.pallas_call(
        _vq_block,
        grid=(b // bb,),
        in_specs=[
            pl.BlockSpec((bb, d), lambda i: (i, 0)),
            pl.BlockSpec((1, k, d), lambda i: (0, 0, 0)),
            pl.BlockSpec((1, k), lambda i: (0, 0)),
        ],
        out_specs=[
            pl.BlockSpec((bb, k), lambda i: (i, 0)),
            pl.BlockSpec((bb, 1), lambda i: (i, 0)),
        ],
        out_shape=[
            jax.ShapeDtypeStruct((b, k), jnp.int8),
            jax.ShapeDtypeStruct((b, 1), jnp.float32),
        ],
        scratch_shapes=[pltpu.VMEM((1, k), jnp.float32)],
        compiler_params=pltpu.CompilerParams(
            dimension_semantics=("arbitrary",)),
    )(xf, anchor_pts, ml2)
    logp_out = logp - logp_k
    return (x, logp_out, mask.astype(jnp.bool_).reshape(b, 1, k))


# speedup vs baseline: 1.3156x; 1.3156x over previous
import jax
import jax.numpy as jnp
from jax.experimental import pallas as pl
from jax.experimental.pallas import tpu as pltpu

_BB = 1024


def _vq_block(x_ref, a_ref, ml_ref, mask_ref, logp_k_ref, a2_ref):
    bb, d = x_ref.shape
    k = a_ref.shape[1]

    @pl.when(pl.program_id(0) == 0)
    def _():
        a0 = a_ref[0]
        a2_ref[...] = jnp.sum(a0 * a0, axis=1)[None, :]

    x = x_ref[...]
    a = a_ref[0]
    xa = jax.lax.dot_general(x * -2.0, a, (((1,), (1,)), ((), ())),
                             preferred_element_type=jnp.float32)
    x2 = jnp.sum(x * x, axis=1, keepdims=True)
    dists = (x2 + a2_ref[...]) + xa
    minv = jnp.min(dists, axis=1, keepdims=True)
    eq = dists == minv
    ml = ml_ref[...]
    shifted = ml - jnp.max(ml)
    lp = shifted - jnp.log(jnp.sum(jnp.exp(shifted)))
    mask_ref[...] = eq.astype(jnp.int8)
    logp_k_ref[...] = jnp.sum(jnp.where(eq, lp, 0.0), axis=1, keepdims=True)
    nsel = jnp.count_nonzero(eq)

    @pl.when(nsel != bb)
    def _fix():
        iota = jax.lax.broadcasted_iota(jnp.int32, (bb, k), 1)
        idx = jnp.min(jnp.where(eq, iota, k), axis=1, keepdims=True)
        m = iota == idx
        mask_ref[...] = m.astype(jnp.int8)
        logp_k_ref[...] = jnp.sum(jnp.where(m, lp, 0.0), axis=1, keepdims=True)


def kernel(x, logp, anchor_pts, mixture_logits):
    b, d = x.shape[0], x.size // x.shape[0]
    xf = x.reshape(b, d)
    k = anchor_pts.shape[1]
    ml2 = mixture_logits.reshape(1, k)
    bb = _BB if b % _BB == 0 else b
    mask, logp_k = pl.pallas_call(
        _vq_block,
        grid=(b // bb,),
        in_specs=[
            pl.BlockSpec((bb, d), lambda i: (i, 0)),
            pl.BlockSpec((1, k, d), lambda i: (0, 0, 0)),
            pl.BlockSpec((1, k), lambda i: (0, 0)),
        ],
        out_specs=[
            pl.BlockSpec((bb, k), lambda i: (i, 0)),
            pl.BlockSpec((bb, 1), lambda i: (i, 0)),
        ],
        out_shape=[
            jax.ShapeDtypeStruct((b, k), jnp.int8),
            jax.ShapeDtypeStruct((b, 1), jnp.float32),
        ],
        scratch_shapes=[pltpu.VMEM((1, k), jnp.float32)],
        compiler_params=pltpu.CompilerParams(
            dimension_semantics=("arbitrary",)),
    )(xf, anchor_pts, ml2)
    logp_out = logp - logp_k
    return (x, logp_out, mask.astype(jnp.bool_).reshape(b, 1, k))


# speedup vs baseline: 1.3460x; 1.0231x over previous
import jax
import jax.numpy as jnp
from jax.experimental import pallas as pl
from jax.experimental.pallas import tpu as pltpu

_BB = 2048


def _vq_block(x_ref, a_ref, ml_ref, mask_ref, logp_k_ref, a2_ref):
    bb, d = x_ref.shape
    k = a_ref.shape[1]

    @pl.when(pl.program_id(0) == 0)
    def _():
        a0 = a_ref[0]
        a2_ref[...] = jnp.sum(a0 * a0, axis=1)[None, :]

    x = x_ref[...]
    a = a_ref[0]
    xa = jax.lax.dot_general(x * -2.0, a, (((1,), (1,)), ((), ())),
                             preferred_element_type=jnp.float32)
    x2 = jnp.sum(x * x, axis=1, keepdims=True)
    dists = (x2 + a2_ref[...]) + xa
    minv = jnp.min(dists, axis=1, keepdims=True)
    eq = dists == minv
    ml = ml_ref[...]
    shifted = ml - jnp.max(ml)
    lp = shifted - jnp.log(jnp.sum(jnp.exp(shifted)))
    mask_ref[...] = eq.astype(jnp.int8)
    logp_k_ref[...] = jnp.sum(jnp.where(eq, lp, 0.0), axis=1, keepdims=True)
    nsel = jnp.count_nonzero(eq)

    @pl.when(nsel != bb)
    def _fix():
        iota = jax.lax.broadcasted_iota(jnp.int32, (bb, k), 1)
        idx = jnp.min(jnp.where(eq, iota, k), axis=1, keepdims=True)
        m = iota == idx
        mask_ref[...] = m.astype(jnp.int8)
        logp_k_ref[...] = jnp.sum(jnp.where(m, lp, 0.0), axis=1, keepdims=True)


def kernel(x, logp, anchor_pts, mixture_logits):
    b, d = x.shape[0], x.size // x.shape[0]
    xf = x.reshape(b, d)
    k = anchor_pts.shape[1]
    ml2 = mixture_logits.reshape(1, k)
    bb = _BB if b % _BB == 0 else b
    mask, logp_k = pl.pallas_call(
        _vq_block,
        grid=(b // bb,),
        in_specs=[
            pl.BlockSpec((bb, d), lambda i: (i, 0)),
            pl.BlockSpec((1, k, d), lambda i: (0, 0, 0)),
            pl.BlockSpec((1, k), lambda i: (0, 0)),
        ],
        out_specs=[
            pl.BlockSpec((bb, k), lambda i: (i, 0)),
            pl.BlockSpec((bb, 1), lambda i: (i, 0)),
        ],
        out_shape=[
            jax.ShapeDtypeStruct((b, k), jnp.int8),
            jax.ShapeDtypeStruct((b, 1), jnp.float32),
        ],
        scratch_shapes=[pltpu.VMEM((1, k), jnp.float32)],
        compiler_params=pltpu.CompilerParams(
            dimension_semantics=("arbitrary",)),
    )(xf, anchor_pts, ml2)
    logp_out = logp - logp_k
    return (x, logp_out, mask.astype(jnp.bool_).reshape(b, 1, k))


# speedup vs baseline: 1.3904x; 1.0330x over previous
import jax
import jax.numpy as jnp
from jax.experimental import pallas as pl
from jax.experimental.pallas import tpu as pltpu

_BB = 4096


def _vq_block(x_ref, a_ref, ml_ref, mask_ref, logp_k_ref, a2_ref):
    bb, d = x_ref.shape
    k = a_ref.shape[1]

    @pl.when(pl.program_id(0) == 0)
    def _():
        a0 = a_ref[0]
        a2_ref[...] = jnp.sum(a0 * a0, axis=1)[None, :]

    x = x_ref[...]
    a = a_ref[0]
    xa = jax.lax.dot_general(x * -2.0, a, (((1,), (1,)), ((), ())),
                             preferred_element_type=jnp.float32)
    x2 = jnp.sum(x * x, axis=1, keepdims=True)
    dists = (x2 + a2_ref[...]) + xa
    minv = jnp.min(dists, axis=1, keepdims=True)
    eq = dists == minv
    ml = ml_ref[...]
    shifted = ml - jnp.max(ml)
    lp = shifted - jnp.log(jnp.sum(jnp.exp(shifted)))
    mask_ref[...] = eq.astype(jnp.int8)
    logp_k_ref[...] = jnp.sum(jnp.where(eq, lp, 0.0), axis=1, keepdims=True)
    nsel = jnp.count_nonzero(eq)

    @pl.when(nsel != bb)
    def _fix():
        iota = jax.lax.broadcasted_iota(jnp.int32, (bb, k), 1)
        idx = jnp.min(jnp.where(eq, iota, k), axis=1, keepdims=True)
        m = iota == idx
        mask_ref[...] = m.astype(jnp.int8)
        logp_k_ref[...] = jnp.sum(jnp.where(m, lp, 0.0), axis=1, keepdims=True)


def kernel(x, logp, anchor_pts, mixture_logits):
    b, d = x.shape[0], x.size // x.shape[0]
    xf = x.reshape(b, d)
    k = anchor_pts.shape[1]
    ml2 = mixture_logits.reshape(1, k)
    bb = _BB if b % _BB == 0 else b
    mask, logp_k = pl.pallas_call(
        _vq_block,
        grid=(b // bb,),
        in_specs=[
            pl.BlockSpec((bb, d), lambda i: (i, 0)),
            pl.BlockSpec((1, k, d), lambda i: (0, 0, 0)),
            pl.BlockSpec((1, k), lambda i: (0, 0)),
        ],
        out_specs=[
            pl.BlockSpec((bb, k), lambda i: (i, 0)),
            pl.BlockSpec((bb, 1), lambda i: (i, 0)),
        ],
        out_shape=[
            jax.ShapeDtypeStruct((b, k), jnp.int8),
            jax.ShapeDtypeStruct((b, 1), jnp.float32),
        ],
        scratch_shapes=[pltpu.VMEM((1, k), jnp.float32)],
        compiler_params=pltpu.CompilerParams(
            dimension_semantics=("arbitrary",)),
    )(xf, anchor_pts, ml2)
    logp_out = logp - logp_k
    return (x, logp_out, mask.astype(jnp.bool_).reshape(b, 1, k))


# speedup vs baseline: 1.6979x; 1.2211x over previous
import jax
import jax.numpy as jnp
from jax.experimental import pallas as pl
from jax.experimental.pallas import tpu as pltpu

_BB = 4096


def _vq_block(x_ref, logp_ref, a_ref, ml_ref, mask_ref, logp_out_ref, a2_ref):
    bb, d = x_ref.shape
    k = a_ref.shape[1]

    @pl.when(pl.program_id(0) == 0)
    def _():
        a0 = a_ref[0]
        a2_ref[...] = jnp.sum(a0 * a0, axis=1)[None, :]

    x = x_ref[...]
    a = a_ref[0]
    xa = jax.lax.dot_general(x * -2.0, a, (((1,), (1,)), ((), ())),
                             preferred_element_type=jnp.float32)
    x2 = jnp.sum(x * x, axis=1, keepdims=True)
    dists = (x2 + a2_ref[...]) + xa
    minv = jnp.min(dists, axis=1, keepdims=True)
    eq = dists == minv
    ml = ml_ref[...]
    shifted = ml - jnp.max(ml)
    lp = shifted - jnp.log(jnp.sum(jnp.exp(shifted)))
    mask_ref[...] = eq.astype(jnp.int8)
    lpk = jnp.sum(jnp.where(eq, lp, 0.0), axis=1, keepdims=True)
    logp_out_ref[...] = logp_ref[...] - jnp.reshape(lpk, (1, bb))
    nsel = jnp.count_nonzero(eq)

    @pl.when(nsel != bb)
    def _fix():
        iota = jax.lax.broadcasted_iota(jnp.int32, (bb, k), 1)
        idx = jnp.min(jnp.where(eq, iota, k), axis=1, keepdims=True)
        m = iota == idx
        mask_ref[...] = m.astype(jnp.int8)
        lpk2 = jnp.sum(jnp.where(m, lp, 0.0), axis=1, keepdims=True)
        logp_out_ref[...] = logp_ref[...] - jnp.reshape(lpk2, (1, bb))


def kernel(x, logp, anchor_pts, mixture_logits):
    b, d = x.shape[0], x.size // x.shape[0]
    xf = x.reshape(b, d)
    k = anchor_pts.shape[1]
    ml2 = mixture_logits.reshape(1, k)
    bb = _BB if b % _BB == 0 else b
    mask, logp_out = pl.pallas_call(
        _vq_block,
        grid=(b // bb,),
        in_specs=[
            pl.BlockSpec((bb, d), lambda i: (i, 0)),
            pl.BlockSpec((1, bb), lambda i: (0, i)),
            pl.BlockSpec((1, k, d), lambda i: (0, 0, 0)),
            pl.BlockSpec((1, k), lambda i: (0, 0)),
        ],
        out_specs=[
            pl.BlockSpec((bb, k), lambda i: (i, 0)),
            pl.BlockSpec((1, bb), lambda i: (0, i)),
        ],
        out_shape=[
            jax.ShapeDtypeStruct((b, k), jnp.int8),
            jax.ShapeDtypeStruct((1, b), jnp.float32),
        ],
        scratch_shapes=[pltpu.VMEM((1, k), jnp.float32)],
        compiler_params=pltpu.CompilerParams(
            dimension_semantics=("arbitrary",)),
    )(xf, logp.reshape(1, b), anchor_pts, ml2)
    return (x, logp_out.reshape(b, 1), mask.astype(jnp.bool_).reshape(b, 1, k))
